# baseline (speedup 1.0000x reference)
import functools

import jax
import jax.numpy as jnp
from jax import lax
from jax.experimental import pallas as pl
from jax.experimental.pallas import tpu as pltpu
from jax.experimental.pallas import tpu_sc as plsc

SEQ = 200
HID = 64
POS_LEN = SEQ + 2
BATCH = 4096
NT_ROWS = 40200
CLS_BASE = 40000

NC = 2
NS = 16
NW = NC * NS
P_PER_W = -(-POS_LEN // NW)

BBLK = 512
NBG = BBLK // 16
NBLK = BATCH // BBLK


def _ln_table_body(table_ref, pos_ref, cls_ref, gamma_ref, beta_ref, out_ref):
    s = pl.program_id(0)
    srow = jnp.minimum(s + 2, POS_LEN - 1)
    posrow = pos_ref[pl.ds(srow, 1), :]
    body = table_ref[...] + posrow
    row0 = jnp.broadcast_to(cls_ref[pl.ds(0, 1), :], (SEQ, HID))
    row1 = jnp.broadcast_to(cls_ref[pl.ds(1, 1), :], (SEQ, HID))
    rr = lax.broadcasted_iota(jnp.int32, (SEQ, HID), 0)
    clsx = jnp.where(rr == 0, row0, jnp.where(rr == 1, row1, 0.0))
    x = jnp.where(s < SEQ, body, clsx)
    mean = jnp.mean(x, axis=-1, keepdims=True)
    var = jnp.mean(jnp.square(x - mean), axis=-1, keepdims=True)
    y = (x - mean) * lax.rsqrt(var + 1e-5)
    out_ref[...] = y * gamma_ref[...] + beta_ref[...]


def _build_norm_table(table, pos2, cls_rows, gamma, beta):
    return pl.pallas_call(
        _ln_table_body,
        grid=(SEQ + 1,),
        in_specs=[
            pl.BlockSpec((SEQ, HID), lambda s: (0, 0)),
            pl.BlockSpec((POS_LEN, HID), lambda s: (0, 0)),
            pl.BlockSpec((2, HID), lambda s: (0, 0)),
            pl.BlockSpec((1, HID), lambda s: (0, 0)),
            pl.BlockSpec((1, HID), lambda s: (0, 0)),
        ],
        out_specs=pl.BlockSpec((SEQ, HID), lambda s: (s, 0)),
        out_shape=jax.ShapeDtypeStruct((NT_ROWS, HID), jnp.float32),
    )(table, pos2, cls_rows, gamma.reshape(1, HID), beta.reshape(1, HID))


def _make_sc_emit():
    mesh = plsc.VectorSubcoreMesh(core_axis_name="c", subcore_axis_name="s")

    @functools.partial(
        pl.kernel,
        mesh=mesh,
        compiler_params=pltpu.CompilerParams(use_tc_tiling_on_sc=True,
                                             needs_layout_passes=False),
        out_type=jax.ShapeDtypeStruct((POS_LEN, HID, BATCH), jnp.float32),
        scratch_types=[
            pltpu.VMEM((SEQ * HID,), jnp.float32),
            pltpu.VMEM((BATCH,), jnp.int32),
            pltpu.VMEM((HID, BBLK), jnp.float32),
            pltpu.VMEM((HID, BBLK), jnp.float32),
            pltpu.SemaphoreType.DMA,
            pltpu.SemaphoreType.DMA,
        ],
    )
    def emit_kernel(nt_hbm, ids_hbm, out_hbm, slab_v, ids_v, ob0, ob1,
                    w0, w1):
        wid = lax.axis_index("s") * NC + lax.axis_index("c")
        obs = (ob0, ob1)
        wsem = (w0, w1)

        def do_position(p):
            slab_base = jnp.where(p < 2, CLS_BASE, (p - 2) * SEQ) * HID
            pltpu.sync_copy(nt_hbm.at[pl.ds(slab_base, SEQ * HID)], slab_v)
            pltpu.sync_copy(ids_hbm.at[p], ids_v)

            def wcopy(blk, slot):
                return pltpu.make_async_copy(
                    obs[slot], out_hbm.at[p, :, pl.ds(blk * BBLK, BBLK)],
                    wsem[slot])

            def fill(blk, slot):
                ob = obs[slot]

                def bg_body(g, carry):
                    col = g * 16
                    idx = ids_v[pl.ds(blk * BBLK + col, 16)]
                    f0 = idx * HID

                    def hg_body(hg, carry2):
                        h0 = hg * 8
                        for hh in range(8):
                            h = h0 + hh
                            v = plsc.load_gather(slab_v, [f0 + h])
                            ob[h, pl.ds(col, 16)] = v
                        return carry2

                    lax.fori_loop(0, HID // 8, hg_body, 0)
                    return carry

                lax.fori_loop(0, NBG, bg_body, 0)

            def pair_body(q, carry):
                blk_a = 2 * q
                blk_b = blk_a + 1

                @pl.when(q > 0)
                def _():
                    wcopy(blk_a - 2, 0).wait()
                fill(blk_a, 0)
                wcopy(blk_a, 0).start()

                @pl.when(q > 0)
                def _():
                    wcopy(blk_b - 2, 1).wait()
                fill(blk_b, 1)
                wcopy(blk_b, 1).start()
                return carry

            lax.fori_loop(0, NBLK // 2, pair_body, 0)
            wcopy(NBLK - 2, 0).wait()
            wcopy(NBLK - 1, 1).wait()

        def p_body(j, carry):
            p = wid + j * NW

            @pl.when(p < POS_LEN)
            def _():
                do_position(p)
            return carry

        lax.fori_loop(0, P_PER_W, p_body, 0)

    return emit_kernel


def kernel(input_ids, table, cls_domain, cls_task, pos_encoding, gamma, beta):
    pos2 = pos_encoding.reshape(POS_LEN, HID)
    cls_rows = jnp.concatenate(
        [cls_domain.reshape(1, HID) + pos2[0:1],
         cls_task.reshape(1, HID) + pos2[1:2]], axis=0)

    nt = _build_norm_table(table, pos2, cls_rows, gamma, beta)
    nt1d = nt.reshape(NT_ROWS * HID)

    ids = input_ids.astype(jnp.int32)
    ids_t = jnp.concatenate(
        [jnp.zeros((1, BATCH), jnp.int32),
         jnp.ones((1, BATCH), jnp.int32),
         ids.T], axis=0)

    out3 = _make_sc_emit()(nt1d, ids_t)
    return jnp.transpose(out3, (2, 0, 1))
